# baseline (speedup 1.0000x reference)
import functools

import jax
import jax.numpy as jnp
from jax.experimental import pallas as pl
from jax.experimental.pallas import tpu as pltpu

TOP_K = 8
NUM_EXPERTS = 64
EMBED_DIM = 4096
BLOCK_T = 1024

_KS0 = 0
_KS1 = 42
_KS2 = 0x1BD11BDA ^ _KS0 ^ _KS1
_ROTS = ((13, 15, 26, 6), (17, 29, 16, 24))
_INJECT = ((_KS1, _KS2, 1), (_KS2, _KS0, 2), (_KS0, _KS1, 3),
           (_KS1, _KS2, 4), (_KS2, _KS0, 5))


def _uniform_bits(flat):
    x0 = jnp.full(flat.shape, jnp.uint32(_KS0), jnp.uint32)
    x1 = flat + jnp.uint32(_KS1)
    for blk in range(5):
        for r in _ROTS[blk % 2]:
            x0 = x0 + x1
            x1 = (x1 << r) | (x1 >> (32 - r))
            x1 = x1 ^ x0
        ka, kb, c = _INJECT[blk]
        x0 = x0 + jnp.uint32(ka)
        x1 = x1 + jnp.uint32(kb + c)
    bits = x0 ^ x1
    return jax.lax.bitcast_convert_type(
        (bits >> 9) | jnp.uint32(0x3F800000), jnp.float32
    ) - 1.0


def _router_block(x_ref, wr_ref, wn_ref, br_ref, bn_ref, out_ref, idx_ref, w_s):
    @pl.when(pl.program_id(0) == 0)
    def _():
        w_s[:, :NUM_EXPERTS] = wr_ref[...]
        w_s[:, NUM_EXPERTS:] = wn_ref[...]

    x = x_ref[...]
    acc = jnp.dot(x, w_s[...], preferred_element_type=jnp.float32)
    acc_t = acc.T
    t = acc.shape[0]
    logits = acc_t[:NUM_EXPERTS, :] + br_ref[...]
    noise_logits = acc_t[NUM_EXPERTS:, :] + bn_ref[...]
    sp = jnp.maximum(noise_logits, 0.0) + jnp.log1p(jnp.exp(-jnp.abs(noise_logits)))
    row = jax.lax.broadcasted_iota(jnp.int32, (NUM_EXPERTS, t), 0)
    col = jax.lax.broadcasted_iota(jnp.int32, (NUM_EXPERTS, t), 1)
    flat = pl.program_id(0) * (BLOCK_T * NUM_EXPERTS) + col * NUM_EXPERTS + row
    u = _uniform_bits(flat.astype(jnp.uint32))
    noisy = logits + u * sp

    iota_f = row.astype(jnp.float32)
    work = noisy
    selected = jnp.zeros((NUM_EXPERTS, t), dtype=jnp.bool_)
    idx_rows = []
    top1 = None
    for j in range(TOP_K):
        m = jnp.max(work, axis=0, keepdims=True)
        if j == 0:
            top1 = m
        idx = jnp.min(
            jnp.where(work == m, iota_f, float(NUM_EXPERTS)), axis=0, keepdims=True
        )
        idx_rows.append(idx)
        hit = iota_f == idx
        selected = jnp.logical_or(selected, hit)
        work = jnp.where(hit, -jnp.inf, work)

    e = jnp.where(selected, jnp.exp(noisy - top1), 0.0)
    out_t = e * (1.0 / jnp.sum(e, axis=0, keepdims=True))
    out_ref[...] = out_t.T
    idx_ref[...] = jnp.concatenate(idx_rows, axis=0).astype(jnp.int32).T


@functools.partial(jax.jit, static_argnames=())
def _run(x, w_route, w_noise, b_route, b_noise):
    n_tok = x.shape[0]
    grid = (n_tok // BLOCK_T,)
    out, idx = pl.pallas_call(
        _router_block,
        grid=grid,
        in_specs=[
            pl.BlockSpec((BLOCK_T, EMBED_DIM), lambda i: (i, 0)),
            pl.BlockSpec((EMBED_DIM, NUM_EXPERTS), lambda i: (0, 0)),
            pl.BlockSpec((EMBED_DIM, NUM_EXPERTS), lambda i: (0, 0)),
            pl.BlockSpec((NUM_EXPERTS, 1), lambda i: (0, 0)),
            pl.BlockSpec((NUM_EXPERTS, 1), lambda i: (0, 0)),
        ],
        out_specs=[
            pl.BlockSpec((BLOCK_T, NUM_EXPERTS), lambda i: (i, 0)),
            pl.BlockSpec((BLOCK_T, TOP_K), lambda i: (i, 0)),
        ],
        out_shape=[
            jax.ShapeDtypeStruct((n_tok, NUM_EXPERTS), jnp.float32),
            jax.ShapeDtypeStruct((n_tok, TOP_K), jnp.int32),
        ],
        scratch_shapes=[pltpu.VMEM((EMBED_DIM, 2 * NUM_EXPERTS), jnp.float32)],
    )(x, w_route, w_noise, b_route, b_noise)
    return out, idx


def kernel(mh_output, W_route, b_route, W_noise, b_noise):
    b, s, d = mh_output.shape
    x = mh_output.reshape(b * s, d)
    out, idx = _run(
        x,
        W_route,
        W_noise,
        b_route.reshape(NUM_EXPERTS, 1),
        b_noise.reshape(NUM_EXPERTS, 1),
    )
    return out.reshape(b, s, NUM_EXPERTS), idx.reshape(b, s, TOP_K)


# speedup vs baseline: 1.0448x; 1.0448x over previous
import functools

import jax
import jax.numpy as jnp
from jax.experimental import pallas as pl

TOP_K = 8
NUM_EXPERTS = 64
EMBED_DIM = 4096
BLOCK_T = 1024

_KS0 = 0
_KS1 = 42
_KS2 = 0x1BD11BDA ^ _KS0 ^ _KS1
_ROTS = ((13, 15, 26, 6), (17, 29, 16, 24))
_INJECT = ((_KS1, _KS2, 1), (_KS2, _KS0, 2), (_KS0, _KS1, 3),
           (_KS1, _KS2, 4), (_KS2, _KS0, 5))


def _uniform_bits(flat):
    x0 = jnp.full(flat.shape, jnp.uint32(_KS0), jnp.uint32)
    x1 = flat + jnp.uint32(_KS1)
    for blk in range(5):
        for r in _ROTS[blk % 2]:
            x0 = x0 + x1
            x1 = (x1 << r) | (x1 >> (32 - r))
            x1 = x1 ^ x0
        ka, kb, c = _INJECT[blk]
        x0 = x0 + jnp.uint32(ka)
        x1 = x1 + jnp.uint32(kb + c)
    bits = x0 ^ x1
    return jax.lax.bitcast_convert_type(
        (bits >> 9) | jnp.uint32(0x3F800000), jnp.float32
    ) - 1.0


def _router_block(x_ref, w_ref, b_ref, out_ref, idx_ref):
    x = x_ref[...]
    w = w_ref[...]
    acc = jnp.dot(x, w, preferred_element_type=jnp.float32) + b_ref[...]
    acc_t = acc.T
    t = acc.shape[0]
    logits = acc_t[:NUM_EXPERTS, :]
    noise_logits = acc_t[NUM_EXPERTS:, :]
    sp = jnp.maximum(noise_logits, 0.0) + jnp.log1p(jnp.exp(-jnp.abs(noise_logits)))
    row = jax.lax.broadcasted_iota(jnp.int32, (NUM_EXPERTS, t), 0)
    col = jax.lax.broadcasted_iota(jnp.int32, (NUM_EXPERTS, t), 1)
    flat = pl.program_id(0) * (BLOCK_T * NUM_EXPERTS) + col * NUM_EXPERTS + row
    u = _uniform_bits(flat.astype(jnp.uint32))
    noisy = logits + u * sp

    iota_f = row.astype(jnp.float32)
    work = noisy
    selected = jnp.zeros((NUM_EXPERTS, t), dtype=jnp.bool_)
    idx_rows = []
    top1 = None
    for j in range(TOP_K):
        m = jnp.max(work, axis=0, keepdims=True)
        if j == 0:
            top1 = m
        idx = jnp.min(
            jnp.where(work == m, iota_f, float(NUM_EXPERTS)), axis=0, keepdims=True
        )
        idx_rows.append(idx)
        hit = iota_f == idx
        selected = jnp.logical_or(selected, hit)
        work = jnp.where(hit, -jnp.inf, work)

    e = jnp.where(selected, jnp.exp(noisy - top1), 0.0)
    out_t = e * (1.0 / jnp.sum(e, axis=0, keepdims=True))
    out_ref[...] = out_t.T
    idx_ref[...] = jnp.concatenate(idx_rows, axis=0).astype(jnp.int32).T


@functools.partial(jax.jit, static_argnames=())
def _run(x, w_cat, b_cat):
    n_tok = x.shape[0]
    grid = (n_tok // BLOCK_T,)
    out, idx = pl.pallas_call(
        _router_block,
        grid=grid,
        in_specs=[
            pl.BlockSpec((BLOCK_T, EMBED_DIM), lambda i: (i, 0)),
            pl.BlockSpec((EMBED_DIM, 2 * NUM_EXPERTS), lambda i: (0, 0)),
            pl.BlockSpec((1, 2 * NUM_EXPERTS), lambda i: (0, 0)),
        ],
        out_specs=[
            pl.BlockSpec((BLOCK_T, NUM_EXPERTS), lambda i: (i, 0)),
            pl.BlockSpec((BLOCK_T, TOP_K), lambda i: (i, 0)),
        ],
        out_shape=[
            jax.ShapeDtypeStruct((n_tok, NUM_EXPERTS), jnp.float32),
            jax.ShapeDtypeStruct((n_tok, TOP_K), jnp.int32),
        ],
    )(x, w_cat, b_cat)
    return out, idx


def kernel(mh_output, W_route, b_route, W_noise, b_noise):
    b, s, d = mh_output.shape
    x = mh_output.reshape(b * s, d)
    w_cat = jnp.concatenate([W_route, W_noise], axis=1)
    b_cat = jnp.concatenate([b_route, b_noise], axis=0).reshape(1, 2 * NUM_EXPERTS)
    out, idx = _run(x, w_cat, b_cat)
    return out.reshape(b, s, NUM_EXPERTS), idx.reshape(b, s, TOP_K)
